# Initial kernel scaffold; baseline (speedup 1.0000x reference)
import jax
import jax.numpy as jnp
from jax.experimental import pallas as pl


def kernel(seq, adj, w, bias, alpha):
    raise NotImplementedError("write your pallas kernel here")


# speedup vs baseline: 1.9358x; 1.9358x over previous
import functools

import jax
import jax.numpy as jnp
from jax.experimental import pallas as pl
from jax.experimental.pallas import tpu as pltpu

LANE = 128
_VMEM_LIMIT = 44 * 1024 * 1024


def _round_up(x, m):
    return (x + m - 1) // m * m


def _pick_tile(dim_p, pref):
    t = max(LANE, min(pref, dim_p))
    t = (t // LANE) * LANE
    while dim_p % t:
        t -= LANE
    return t


def _gcn_body(alpha_ref, adj_ref, seq_ref, w_ref, bias_ref, o_ref, xw_ref):
    @pl.when(pl.program_id(1) == 0)
    def _():
        xw_ref[...] = jnp.dot(
            seq_ref[...].astype(jnp.bfloat16), w_ref[...],
            preferred_element_type=jnp.float32).astype(jnp.bfloat16)

    h = jnp.dot(adj_ref[...].astype(jnp.bfloat16), xw_ref[...],
                preferred_element_type=jnp.float32) + bias_ref[...]
    alpha = alpha_ref[0]
    o_ref[...] = jnp.where(h > 0.0, h, alpha * h)


@jax.jit
def kernel(seq, adj, w, bias, alpha):
    B, N, F_in = seq.shape
    F_h = w.shape[1]
    alpha1d = jnp.asarray(alpha, jnp.float32).reshape(1)

    Np = _round_up(N, LANE)
    Fi = _round_up(F_in, LANE)
    Fh = _round_up(F_h, LANE)
    seq_p = jnp.pad(seq.astype(jnp.float32), ((0, 0), (0, Np - N), (0, Fi - F_in)))
    adj_p = jnp.pad(adj.astype(jnp.float32), ((0, 0), (0, Np - N), (0, Np - N)))
    w_p = jnp.pad(w, ((0, Fi - F_in), (0, Fh - F_h))).astype(jnp.bfloat16)
    bias_p = jnp.pad(bias, (0, Fh - F_h)).reshape(1, Fh).astype(jnp.float32)

    tn = _pick_tile(Np, 512)

    def _vmem_bytes(tn_):
        return (2 * tn_ * Np * 4
                + 2 * Np * Fi * 4
                + 2 * tn_ * Fh * 4
                + Np * Fh * 2
                + Np * Fh * 4
                + 2 * Fi * Fh * 2)

    while _vmem_bytes(tn) > _VMEM_LIMIT - 2 * 1024 * 1024 and tn > LANE:
        tn = _pick_tile(Np, tn - LANE)

    grid = (B, Np // tn)
    flops = 2 * B * (Np * Fi * Fh + Np * Np * Fh)
    bytes_accessed = (B * (Np * Np + Np * Fi + Np * Fh) * 4
                      + Fi * Fh * 2 + Fh * 4)

    out = pl.pallas_call(
        _gcn_body,
        out_shape=jax.ShapeDtypeStruct((B, Np, Fh), jnp.float32),
        grid=grid,
        in_specs=[
            pl.BlockSpec(memory_space=pltpu.MemorySpace.SMEM),
            pl.BlockSpec((None, tn, Np), lambda b, i: (b, i, 0)),
            pl.BlockSpec((None, Np, Fi), lambda b, i: (b, 0, 0)),
            pl.BlockSpec((Fi, Fh), lambda b, i: (0, 0)),
            pl.BlockSpec((1, Fh), lambda b, i: (0, 0)),
        ],
        out_specs=pl.BlockSpec((None, tn, Fh), lambda b, i: (b, i, 0)),
        scratch_shapes=[pltpu.VMEM((Np, Fh), jnp.bfloat16)],
        compiler_params=pltpu.CompilerParams(
            dimension_semantics=("parallel", "arbitrary"),
            vmem_limit_bytes=_VMEM_LIMIT),
        cost_estimate=pl.CostEstimate(flops=flops, transcendentals=0,
                                      bytes_accessed=bytes_accessed),
    )(alpha1d, adj_p, seq_p, w_p, bias_p)
    return out[:, :N, :F_h]
